# baseline (speedup 1.0000x reference)
import functools
import math

import jax
import jax.numpy as jnp
from jax import lax
from jax.experimental import pallas as pl
from jax.experimental.pallas import tpu as pltpu
from jax.experimental.pallas import tpu_sc as plsc

N = 10000
NPAD = 10240
E = 10000
M = 160000
DIM_IN = 128
DIM_OUT = 64
HEADS = 4
DS = 16
NI = 4
UW = 144
KW = 4 * DIM_OUT

_NC = 2
_NS = 16
MP = 163840
CH = 128
TPS = MP // (CH * _NS)
TPW = MP // (CH * _NC * _NS)
ESH = E + 16
QW = 128

_SCALE = 1.0 / math.sqrt(DIM_OUT)


def _iota2(shape, dim):
    return lax.broadcasted_iota(jnp.int32, shape, dim)


def _onehot_f32(pred):
    return pred.astype(jnp.float32)


def _dotT(a, w):
    return lax.dot_general(a, w, (((1,), (1,)), ((), ())),
                           preferred_element_type=jnp.float32)


def _dot(a, w):
    return lax.dot_general(a, w, (((1,), (0,)), ((), ())),
                           preferred_element_type=jnp.float32)


def _ln(o, g, b):
    mu = jnp.mean(o, axis=1, keepdims=True)
    var = jnp.mean((o - mu) ** 2, axis=1, keepdims=True)
    return (o - mu) * lax.rsqrt(var + 1e-5) * g + b


def _perm_mat():
    i = _iota2((DIM_OUT, DIM_OUT), 0)
    j = _iota2((DIM_OUT, DIM_OUT), 1)
    return _onehot_f32(j == (i % DS) * HEADS + i // DS)


def _pre_body(x_ref, i_ref, wq0_ref, bq0_ref, wk0_ref, bk0_ref,
              wv0_ref, bv0_ref, wq1_ref, bq1_ref,
              ua_ref, ub_ref, qp1_ref):
    X = x_ref[...]
    Kp = _dotT(X, wk0_ref[...]) + bk0_ref[...]
    Vp = _dotT(X, wv0_ref[...]) + bv0_ref[...]
    Qp1 = _dotT(X, wq1_ref[...]) + bq1_ref[...]
    qp1_ref[...] = jnp.concatenate(
        [Qp1, jnp.zeros((X.shape[0], QW - DIM_OUT), jnp.float32)], axis=1)
    Qind = _dotT(i_ref[...], wq0_ref[...]) + bq0_ref[...]

    hmap = _onehot_f32(_iota2((DIM_OUT, HEADS), 0) // DS == _iota2((DIM_OUT, HEADS), 1))
    parts = []
    for inc in range(NI):
        parts.append(_dot(Kp * Qind[inc:inc + 1, :], hmap))
    S = jnp.concatenate(parts, axis=1) * _SCALE

    rows = _iota2(S.shape, 0)
    gmax = jnp.max(jnp.where(rows < N, S, -jnp.inf), axis=0, keepdims=True)
    Ex = jnp.exp(S - gmax)

    r8 = _onehot_f32(_iota2((8, 128), 1) // DS == _iota2((8, 128), 0))
    t2 = _onehot_f32(_iota2((DIM_OUT, 128), 1) % DIM_OUT == _iota2((DIM_OUT, 128), 0))
    Vt = _dot(Vp, t2)
    zpad = jnp.zeros((X.shape[0], 8), jnp.float32)
    for c, out in ((0, ua_ref), (1, ub_ref)):
        Eh = Ex[:, 8 * c:8 * c + 8]
        Eexp = _dot(Eh, r8)
        out[...] = jnp.concatenate([Eexp * Vt, Eh, zpad], axis=1)


def _stage1(Xp, Ipad, Wq0, bq0r, Wk0, bk0r, Wv0, bv0r, Wq1, bq1r):
    return pl.pallas_call(
        _pre_body,
        out_shape=[
            jax.ShapeDtypeStruct((NPAD, UW), jnp.float32),
            jax.ShapeDtypeStruct((NPAD, UW), jnp.float32),
            jax.ShapeDtypeStruct((NPAD, QW), jnp.float32),
        ],
    )(Xp, Ipad, Wq0, bq0r, Wk0, bk0r, Wv0, bv0r, Wq1, bq1r)


def _scatter_body(nidx, eidx, ua, ub, zinit, acc_a, acc_b,
                  nbuf, ebuf, rows, shared, sem):
    cid = lax.axis_index("c")
    sid = lax.axis_index("s")

    @pl.when(sid == 0)
    def _():
        pltpu.sync_copy(zinit, shared)

    plsc.subcore_barrier()

    def run(table, acc):
        base0 = sid * TPS

        def body(t, carry):
            base = (base0 + t) * CH
            pltpu.sync_copy(nidx.at[pl.ds(base, CH)], nbuf)
            pltpu.sync_copy(eidx.at[pl.ds(base, CH)], ebuf)
            pltpu.async_copy(table.at[nbuf], rows, sem).wait()
            pltpu.sync_copy(rows, shared.at[ebuf], add=True)
            return carry

        lax.fori_loop(0, TPS, body, 0)
        plsc.subcore_barrier()

        @pl.when(sid == 0)
        def _():
            pltpu.sync_copy(shared.at[pl.ds(0, E)], acc)

    @pl.when(cid == 0)
    def _():
        run(ua, acc_a)

    @pl.when(cid == 1)
    def _():
        run(ub, acc_b)


def _mab1_scatter(nidx, eidx, ua, ub, zinit):
    return pl.kernel(
        _scatter_body,
        out_type=[
            jax.ShapeDtypeStruct((E, UW), jnp.float32),
            jax.ShapeDtypeStruct((E, UW), jnp.float32),
        ],
        mesh=plsc.VectorSubcoreMesh(core_axis_name="c", subcore_axis_name="s"),
        compiler_params=pltpu.CompilerParams(use_tc_tiling_on_sc=False),
        scratch_types=[
            pltpu.VMEM((CH,), jnp.int32),
            pltpu.VMEM((CH,), jnp.int32),
            pltpu.VMEM((CH, UW), jnp.float32),
            pltpu.VMEM_SHARED((ESH, UW), jnp.float32),
            pltpu.SemaphoreType.DMA,
        ],
    )(nidx, eidx, ua, ub, zinit)


def _mid_body(acc_a_ref, acc_b_ref, i_ref, wq0_ref, bq0_ref,
              wo0_ref, bo0_ref, g00_ref, be00_ref, g01_ref, be01_ref,
              wk1_ref, bk1_ref, wv1_ref, bv1_ref,
              klo_ref, khi_ref, vlo_ref, vhi_ref):
    Qind = _dotT(i_ref[...], wq0_ref[...]) + bq0_ref[...]
    permM = _perm_mat()
    r4 = _onehot_f32(_iota2((HEADS, DIM_OUT), 1) // DS == _iota2((HEADS, DIM_OUT), 0))
    srcs = (acc_a_ref[...], acc_b_ref[...])
    pk, pv = [], []
    for g in range(NI):
        src = srcs[g // 2]
        lc = g % 2
        Num = src[:, DIM_OUT * lc:DIM_OUT * lc + DIM_OUT]
        Den = _dot(src[:, 128 + 4 * lc:132 + 4 * lc], r4)
        QKV = jnp.where(Den > 0, Num / Den, 0.0)
        O = QKV + Qind[g:g + 1, :]
        O = _dot(O, permM)
        O = _ln(O, g00_ref[...], be00_ref[...])
        O = O + jnp.maximum(_dotT(O, wo0_ref[...]) + bo0_ref[...], 0.0)
        O = _ln(O, g01_ref[...], be01_ref[...])
        pk.append(_dotT(O, wk1_ref[...]) + bk1_ref[...])
        pv.append(_dotT(O, wv1_ref[...]) + bv1_ref[...])
    klo_ref[...] = jnp.concatenate(pk[:2], axis=1)
    khi_ref[...] = jnp.concatenate(pk[2:], axis=1)
    vlo_ref[...] = jnp.concatenate(pv[:2], axis=1)
    vhi_ref[...] = jnp.concatenate(pv[2:], axis=1)


def _stage3(acc_a, acc_b, Ipad, Wq0, bq0r, Wo0, bo0r, g00r, be00r, g01r,
            be01r, Wk1, bk1r, Wv1, bv1r):
    BLK = 2000
    grid = E // BLK
    full = lambda s: pl.BlockSpec(s, lambda i: (0, 0))
    return pl.pallas_call(
        _mid_body,
        grid=(grid,),
        in_specs=[
            pl.BlockSpec((BLK, UW), lambda i: (i, 0)),
            pl.BlockSpec((BLK, UW), lambda i: (i, 0)),
            full((8, DIM_OUT)), full((DIM_OUT, DIM_OUT)), full((1, DIM_OUT)),
            full((DIM_OUT, DIM_OUT)), full((1, DIM_OUT)),
            full((1, DIM_OUT)), full((1, DIM_OUT)), full((1, DIM_OUT)), full((1, DIM_OUT)),
            full((DIM_OUT, DIM_OUT)), full((1, DIM_OUT)),
            full((DIM_OUT, DIM_OUT)), full((1, DIM_OUT)),
        ],
        out_specs=[pl.BlockSpec((BLK, QW), lambda i: (i, 0))] * 4,
        out_shape=[jax.ShapeDtypeStruct((E, QW), jnp.float32)] * 4,
    )(acc_a, acc_b, Ipad, Wq0, bq0r, Wo0, bo0r, g00r, be00r, g01r, be01r,
      Wk1, bk1r, Wv1, bv1r)


def _gather_body(nidx, eidx, qtab, klo, khi, vlo, vhi,
                 g1, g2a, g2b, g3a, g3b,
                 nbuf, ebuf, qrows, karows, kbrows, varows, vbrows,
                 s1, s2, s3, s4, s5):
    cid = lax.axis_index("c")
    sid = lax.axis_index("s")
    wid = sid * _NC + cid
    base0 = wid * TPW

    def body(t, carry):
        base = (base0 + t) * CH
        pltpu.sync_copy(nidx.at[pl.ds(base, CH)], nbuf)
        pltpu.sync_copy(eidx.at[pl.ds(base, CH)], ebuf)
        c1 = pltpu.async_copy(qtab.at[nbuf], qrows, s1)
        c2 = pltpu.async_copy(klo.at[ebuf], karows, s2)
        c3 = pltpu.async_copy(khi.at[ebuf], kbrows, s3)
        c4 = pltpu.async_copy(vlo.at[ebuf], varows, s4)
        c5 = pltpu.async_copy(vhi.at[ebuf], vbrows, s5)
        c1.wait()
        c2.wait()
        c3.wait()
        c4.wait()
        c5.wait()
        pltpu.sync_copy(qrows, g1.at[pl.ds(base, CH)])
        pltpu.sync_copy(karows, g2a.at[pl.ds(base, CH)])
        pltpu.sync_copy(kbrows, g2b.at[pl.ds(base, CH)])
        pltpu.sync_copy(varows, g3a.at[pl.ds(base, CH)])
        pltpu.sync_copy(vbrows, g3b.at[pl.ds(base, CH)])
        return carry

    lax.fori_loop(0, TPW, body, 0)


def _mab2_gather(nidx, eidx, qp1, klo, khi, vlo, vhi):
    return pl.kernel(
        _gather_body,
        out_type=[jax.ShapeDtypeStruct((MP, QW), jnp.float32)] * 5,
        mesh=plsc.VectorSubcoreMesh(core_axis_name="c", subcore_axis_name="s"),
        compiler_params=pltpu.CompilerParams(use_tc_tiling_on_sc=False),
        scratch_types=[
            pltpu.VMEM((CH,), jnp.int32),
            pltpu.VMEM((CH,), jnp.int32),
            pltpu.VMEM((CH, QW), jnp.float32),
            pltpu.VMEM((CH, QW), jnp.float32),
            pltpu.VMEM((CH, QW), jnp.float32),
            pltpu.VMEM((CH, QW), jnp.float32),
            pltpu.VMEM((CH, QW), jnp.float32),
            pltpu.SemaphoreType.DMA,
            pltpu.SemaphoreType.DMA,
            pltpu.SemaphoreType.DMA,
            pltpu.SemaphoreType.DMA,
            pltpu.SemaphoreType.DMA,
        ],
    )(nidx, eidx, qp1, klo, khi, vlo, vhi)


def _post_body(g1_ref, g2a_ref, g2b_ref, g3a_ref, g3b_ref, wo1_ref, bo1_ref,
               g10_ref, be10_ref, g11_ref, be11_ref, out_ref):
    q = g1_ref[:, :DIM_OUT]
    klo = g2a_ref[...]
    khi = g2b_ref[...]
    vlo = g3a_ref[...]
    vhi = g3b_ref[...]
    t2h = _onehot_f32(_iota2((DIM_OUT, QW), 1) % DIM_OUT == _iota2((DIM_OUT, QW), 0))
    qt = _dot(q, t2h)
    cgrp = _iota2((QW, 16), 0) // DS
    ccol = _iota2((QW, 16), 1)
    msumL = _onehot_f32(ccol == cgrp)
    msumH = _onehot_f32(ccol == cgrp + 8)
    A = (_dot(qt * klo, msumL) + _dot(qt * khi, msumH)) * _SCALE
    ap = [A[:, 4 * i:4 * i + 4] for i in range(NI)]
    mx = jnp.maximum(jnp.maximum(ap[0], ap[1]), jnp.maximum(ap[2], ap[3]))
    es = [jnp.exp(p - mx) for p in ap]
    den = es[0] + es[1] + es[2] + es[3]
    w = jnp.concatenate([e / den for e in es], axis=1)
    rgrp = _iota2((16, QW), 1) // DS
    rrow = _iota2((16, QW), 0)
    r16L = _onehot_f32(rrow == rgrp)
    r16H = _onehot_f32(rrow == rgrp + 8)
    m64 = _onehot_f32(_iota2((QW, DIM_OUT), 0) % DIM_OUT == _iota2((QW, DIM_OUT), 1))
    attn = _dot(_dot(w, r16L) * vlo, m64) + _dot(_dot(w, r16H) * vhi, m64)
    O = q + attn
    O = _dot(O, _perm_mat())
    O = _ln(O, g10_ref[...], be10_ref[...])
    O = O + jnp.maximum(_dotT(O, wo1_ref[...]) + bo1_ref[...], 0.0)
    out_ref[...] = _ln(O, g11_ref[...], be11_ref[...])


def _stage5(G1, G2a, G2b, G3a, G3b, Wo1, bo1r, g10r, be10r, g11r, be11r):
    BLK = 4096
    grid = MP // BLK
    full = lambda s: pl.BlockSpec(s, lambda i: (0, 0))
    return pl.pallas_call(
        _post_body,
        grid=(grid,),
        in_specs=[pl.BlockSpec((BLK, QW), lambda i: (i, 0))] * 5 + [
            full((DIM_OUT, DIM_OUT)), full((1, DIM_OUT)),
            full((1, DIM_OUT)), full((1, DIM_OUT)), full((1, DIM_OUT)), full((1, DIM_OUT)),
        ],
        out_specs=pl.BlockSpec((BLK, DIM_OUT), lambda i: (i, 0)),
        out_shape=jax.ShapeDtypeStruct((MP, DIM_OUT), jnp.float32),
    )(G1, G2a, G2b, G3a, G3b, Wo1, bo1r, g10r, be10r, g11r, be11r)


def kernel(X, hyperedge_index, I, Wq0, bq0, Wk0, bk0, Wv0, bv0, Wo0, bo0,
           g00, be00, g01, be01, Wq1, bq1, Wk1, bk1, Wv1, bv1, Wo1, bo1,
           g10, be10, g11, be11, data, name):
    row = lambda b: b.reshape(1, -1)
    Xp = jnp.pad(X, ((0, NPAD - N), (0, 0)))
    Ipad = jnp.pad(I, ((0, 8 - NI), (0, 0)))
    nidx = jnp.pad(hyperedge_index[0], (0, MP - M))
    eidx2 = jnp.concatenate(
        [hyperedge_index[1],
         E + (jnp.arange(MP - M, dtype=jnp.int32) % 16)])
    eidx4 = jnp.pad(hyperedge_index[1], (0, MP - M))

    Ua, Ub, Qp1 = _stage1(Xp, Ipad, Wq0, row(bq0), Wk0, row(bk0),
                          Wv0, row(bv0), Wq1, row(bq1))
    zinit = jnp.zeros((ESH, UW), jnp.float32)
    Acc_a, Acc_b = _mab1_scatter(nidx, eidx2, Ua, Ub, zinit)
    Klo, Khi, Vlo, Vhi = _stage3(Acc_a, Acc_b, Ipad, Wq0, row(bq0), Wo0,
                                 row(bo0), row(g00), row(be00), row(g01),
                                 row(be01), Wk1, row(bk1), Wv1, row(bv1))
    G1, G2a, G2b, G3a, G3b = _mab2_gather(nidx, eidx4, Qp1, Klo, Khi, Vlo, Vhi)
    out = _stage5(G1, G2a, G2b, G3a, G3b, Wo1, row(bo1), row(g10), row(be10),
                  row(g11), row(be11))
    return out[:M]


# speedup vs baseline: 1.1055x; 1.1055x over previous
import functools
import math

import jax
import jax.numpy as jnp
from jax import lax
from jax.experimental import pallas as pl
from jax.experimental.pallas import tpu as pltpu
from jax.experimental.pallas import tpu_sc as plsc

N = 10000
NPAD = 10240
E = 10000
M = 160000
DIM_IN = 128
DIM_OUT = 64
HEADS = 4
DS = 16
NI = 4
UW = 144
KW = 4 * DIM_OUT

_NC = 2
_NS = 16
MP = 163840
CH = 128
TPS = MP // (CH * _NS)
TPW = MP // (CH * _NC * _NS)
ESH = E + 16
QW = 128

_SCALE = 1.0 / math.sqrt(DIM_OUT)


def _iota2(shape, dim):
    return lax.broadcasted_iota(jnp.int32, shape, dim)


def _onehot_f32(pred):
    return pred.astype(jnp.float32)


def _dotT(a, w):
    return lax.dot_general(a, w, (((1,), (1,)), ((), ())),
                           preferred_element_type=jnp.float32)


def _dot(a, w):
    return lax.dot_general(a, w, (((1,), (0,)), ((), ())),
                           preferred_element_type=jnp.float32)


def _ln(o, g, b):
    mu = jnp.mean(o, axis=1, keepdims=True)
    var = jnp.mean((o - mu) ** 2, axis=1, keepdims=True)
    return (o - mu) * lax.rsqrt(var + 1e-5) * g + b


def _perm_mat():
    i = _iota2((DIM_OUT, DIM_OUT), 0)
    j = _iota2((DIM_OUT, DIM_OUT), 1)
    return _onehot_f32(j == (i % DS) * HEADS + i // DS)


def _pre_body(x_ref, i_ref, wq0_ref, bq0_ref, wk0_ref, bk0_ref,
              wv0_ref, bv0_ref, wq1_ref, bq1_ref,
              ua_ref, ub_ref, qp1_ref):
    X = x_ref[...]
    Kp = _dotT(X, wk0_ref[...]) + bk0_ref[...]
    Vp = _dotT(X, wv0_ref[...]) + bv0_ref[...]
    Qp1 = _dotT(X, wq1_ref[...]) + bq1_ref[...]
    qp1_ref[...] = jnp.concatenate(
        [Qp1, jnp.zeros((X.shape[0], QW - DIM_OUT), jnp.float32)], axis=1)
    Qind = _dotT(i_ref[...], wq0_ref[...]) + bq0_ref[...]

    hmap = _onehot_f32(_iota2((DIM_OUT, HEADS), 0) // DS == _iota2((DIM_OUT, HEADS), 1))
    parts = []
    for inc in range(NI):
        parts.append(_dot(Kp * Qind[inc:inc + 1, :], hmap))
    S = jnp.concatenate(parts, axis=1) * _SCALE

    rows = _iota2(S.shape, 0)
    gmax = jnp.max(jnp.where(rows < N, S, -jnp.inf), axis=0, keepdims=True)
    Ex = jnp.exp(S - gmax)

    r8 = _onehot_f32(_iota2((8, 128), 1) // DS == _iota2((8, 128), 0))
    t2 = _onehot_f32(_iota2((DIM_OUT, 128), 1) % DIM_OUT == _iota2((DIM_OUT, 128), 0))
    Vt = _dot(Vp, t2)
    zpad = jnp.zeros((X.shape[0], 8), jnp.float32)
    for c, out in ((0, ua_ref), (1, ub_ref)):
        Eh = Ex[:, 8 * c:8 * c + 8]
        Eexp = _dot(Eh, r8)
        out[...] = jnp.concatenate([Eexp * Vt, Eh, zpad], axis=1)


def _stage1(Xp, Ipad, Wq0, bq0r, Wk0, bk0r, Wv0, bv0r, Wq1, bq1r):
    return pl.pallas_call(
        _pre_body,
        out_shape=[
            jax.ShapeDtypeStruct((NPAD, UW), jnp.float32),
            jax.ShapeDtypeStruct((NPAD, UW), jnp.float32),
            jax.ShapeDtypeStruct((NPAD, QW), jnp.float32),
        ],
    )(Xp, Ipad, Wq0, bq0r, Wk0, bk0r, Wv0, bv0r, Wq1, bq1r)


def _scatter_body(nidx, eidx, ua, ub, zinit, acc_a, acc_b,
                  nbuf, ebuf, rows, shared, sem):
    cid = lax.axis_index("c")
    sid = lax.axis_index("s")

    @pl.when(sid == 0)
    def _():
        pltpu.sync_copy(zinit, shared)

    plsc.subcore_barrier()

    def run(table, acc):
        def body(t, carry):
            base = (sid + t * _NS) * CH
            pltpu.sync_copy(nidx.at[pl.ds(base, CH)], nbuf)
            pltpu.sync_copy(eidx.at[pl.ds(base, CH)], ebuf)
            pltpu.async_copy(table.at[nbuf], rows, sem).wait()
            pltpu.sync_copy(rows, shared.at[ebuf], add=True)
            return carry

        lax.fori_loop(0, TPS, body, 0)
        plsc.subcore_barrier()

        @pl.when(sid == 0)
        def _():
            pltpu.sync_copy(shared.at[pl.ds(0, E)], acc)

    @pl.when(cid == 0)
    def _():
        run(ua, acc_a)

    @pl.when(cid == 1)
    def _():
        run(ub, acc_b)


def _mab1_scatter(nidx, eidx, ua, ub, zinit):
    return pl.kernel(
        _scatter_body,
        out_type=[
            jax.ShapeDtypeStruct((E, UW), jnp.float32),
            jax.ShapeDtypeStruct((E, UW), jnp.float32),
        ],
        mesh=plsc.VectorSubcoreMesh(core_axis_name="c", subcore_axis_name="s"),
        compiler_params=pltpu.CompilerParams(use_tc_tiling_on_sc=False),
        scratch_types=[
            pltpu.VMEM((CH,), jnp.int32),
            pltpu.VMEM((CH,), jnp.int32),
            pltpu.VMEM((CH, UW), jnp.float32),
            pltpu.VMEM_SHARED((ESH, UW), jnp.float32),
            pltpu.SemaphoreType.DMA,
        ],
    )(nidx, eidx, ua, ub, zinit)


def _mid_body(acc_a_ref, acc_b_ref, i_ref, wq0_ref, bq0_ref,
              wo0_ref, bo0_ref, g00_ref, be00_ref, g01_ref, be01_ref,
              wk1_ref, bk1_ref, wv1_ref, bv1_ref,
              klo_ref, khi_ref, vlo_ref, vhi_ref):
    Qind = _dotT(i_ref[...], wq0_ref[...]) + bq0_ref[...]
    permM = _perm_mat()
    r4 = _onehot_f32(_iota2((HEADS, DIM_OUT), 1) // DS == _iota2((HEADS, DIM_OUT), 0))
    srcs = (acc_a_ref[...], acc_b_ref[...])
    pk, pv = [], []
    for g in range(NI):
        src = srcs[g // 2]
        lc = g % 2
        Num = src[:, DIM_OUT * lc:DIM_OUT * lc + DIM_OUT]
        Den = _dot(src[:, 128 + 4 * lc:132 + 4 * lc], r4)
        QKV = jnp.where(Den > 0, Num / Den, 0.0)
        O = QKV + Qind[g:g + 1, :]
        O = _dot(O, permM)
        O = _ln(O, g00_ref[...], be00_ref[...])
        O = O + jnp.maximum(_dotT(O, wo0_ref[...]) + bo0_ref[...], 0.0)
        O = _ln(O, g01_ref[...], be01_ref[...])
        pk.append(_dotT(O, wk1_ref[...]) + bk1_ref[...])
        pv.append(_dotT(O, wv1_ref[...]) + bv1_ref[...])
    klo_ref[...] = jnp.concatenate(pk[:2], axis=1)
    khi_ref[...] = jnp.concatenate(pk[2:], axis=1)
    vlo_ref[...] = jnp.concatenate(pv[:2], axis=1)
    vhi_ref[...] = jnp.concatenate(pv[2:], axis=1)


def _stage3(acc_a, acc_b, Ipad, Wq0, bq0r, Wo0, bo0r, g00r, be00r, g01r,
            be01r, Wk1, bk1r, Wv1, bv1r):
    BLK = 2000
    grid = E // BLK
    full = lambda s: pl.BlockSpec(s, lambda i: (0, 0))
    return pl.pallas_call(
        _mid_body,
        grid=(grid,),
        in_specs=[
            pl.BlockSpec((BLK, UW), lambda i: (i, 0)),
            pl.BlockSpec((BLK, UW), lambda i: (i, 0)),
            full((8, DIM_OUT)), full((DIM_OUT, DIM_OUT)), full((1, DIM_OUT)),
            full((DIM_OUT, DIM_OUT)), full((1, DIM_OUT)),
            full((1, DIM_OUT)), full((1, DIM_OUT)), full((1, DIM_OUT)), full((1, DIM_OUT)),
            full((DIM_OUT, DIM_OUT)), full((1, DIM_OUT)),
            full((DIM_OUT, DIM_OUT)), full((1, DIM_OUT)),
        ],
        out_specs=[pl.BlockSpec((BLK, QW), lambda i: (i, 0))] * 4,
        out_shape=[jax.ShapeDtypeStruct((E, QW), jnp.float32)] * 4,
    )(acc_a, acc_b, Ipad, Wq0, bq0r, Wo0, bo0r, g00r, be00r, g01r, be01r,
      Wk1, bk1r, Wv1, bv1r)


def _gather_body(nidx, eidx, qtab, klo, khi, vlo, vhi,
                 g1, g2a, g2b, g3a, g3b,
                 nbuf, ebuf, qrows, karows, kbrows, varows, vbrows,
                 s1, s2, s3, s4, s5):
    cid = lax.axis_index("c")
    sid = lax.axis_index("s")
    wid = sid * _NC + cid

    def body(t, carry):
        base = (wid + t * (_NC * _NS)) * CH
        pltpu.sync_copy(nidx.at[pl.ds(base, CH)], nbuf)
        pltpu.sync_copy(eidx.at[pl.ds(base, CH)], ebuf)
        c1 = pltpu.async_copy(qtab.at[nbuf], qrows, s1)
        c2 = pltpu.async_copy(klo.at[ebuf], karows, s2)
        c3 = pltpu.async_copy(khi.at[ebuf], kbrows, s3)
        c4 = pltpu.async_copy(vlo.at[ebuf], varows, s4)
        c5 = pltpu.async_copy(vhi.at[ebuf], vbrows, s5)
        c1.wait()
        c2.wait()
        c3.wait()
        c4.wait()
        c5.wait()
        pltpu.sync_copy(qrows, g1.at[pl.ds(base, CH)])
        pltpu.sync_copy(karows, g2a.at[pl.ds(base, CH)])
        pltpu.sync_copy(kbrows, g2b.at[pl.ds(base, CH)])
        pltpu.sync_copy(varows, g3a.at[pl.ds(base, CH)])
        pltpu.sync_copy(vbrows, g3b.at[pl.ds(base, CH)])
        return carry

    lax.fori_loop(0, TPW, body, 0)


def _mab2_gather(nidx, eidx, qp1, klo, khi, vlo, vhi):
    return pl.kernel(
        _gather_body,
        out_type=[jax.ShapeDtypeStruct((MP, QW), jnp.float32)] * 5,
        mesh=plsc.VectorSubcoreMesh(core_axis_name="c", subcore_axis_name="s"),
        compiler_params=pltpu.CompilerParams(use_tc_tiling_on_sc=False),
        scratch_types=[
            pltpu.VMEM((CH,), jnp.int32),
            pltpu.VMEM((CH,), jnp.int32),
            pltpu.VMEM((CH, QW), jnp.float32),
            pltpu.VMEM((CH, QW), jnp.float32),
            pltpu.VMEM((CH, QW), jnp.float32),
            pltpu.VMEM((CH, QW), jnp.float32),
            pltpu.VMEM((CH, QW), jnp.float32),
            pltpu.SemaphoreType.DMA,
            pltpu.SemaphoreType.DMA,
            pltpu.SemaphoreType.DMA,
            pltpu.SemaphoreType.DMA,
            pltpu.SemaphoreType.DMA,
        ],
    )(nidx, eidx, qp1, klo, khi, vlo, vhi)


def _post_body(g1_ref, g2a_ref, g2b_ref, g3a_ref, g3b_ref, wo1_ref, bo1_ref,
               g10_ref, be10_ref, g11_ref, be11_ref, out_ref):
    q = g1_ref[:, :DIM_OUT]
    klo = g2a_ref[...]
    khi = g2b_ref[...]
    vlo = g3a_ref[...]
    vhi = g3b_ref[...]
    t2h = _onehot_f32(_iota2((DIM_OUT, QW), 1) % DIM_OUT == _iota2((DIM_OUT, QW), 0))
    qt = _dot(q, t2h)
    cgrp = _iota2((QW, 16), 0) // DS
    ccol = _iota2((QW, 16), 1)
    msumL = _onehot_f32(ccol == cgrp)
    msumH = _onehot_f32(ccol == cgrp + 8)
    A = (_dot(qt * klo, msumL) + _dot(qt * khi, msumH)) * _SCALE
    ap = [A[:, 4 * i:4 * i + 4] for i in range(NI)]
    mx = jnp.maximum(jnp.maximum(ap[0], ap[1]), jnp.maximum(ap[2], ap[3]))
    es = [jnp.exp(p - mx) for p in ap]
    den = es[0] + es[1] + es[2] + es[3]
    w = jnp.concatenate([e / den for e in es], axis=1)
    rgrp = _iota2((16, QW), 1) // DS
    rrow = _iota2((16, QW), 0)
    r16L = _onehot_f32(rrow == rgrp)
    r16H = _onehot_f32(rrow == rgrp + 8)
    m64 = _onehot_f32(_iota2((QW, DIM_OUT), 0) % DIM_OUT == _iota2((QW, DIM_OUT), 1))
    attn = _dot(_dot(w, r16L) * vlo, m64) + _dot(_dot(w, r16H) * vhi, m64)
    O = q + attn
    O = _dot(O, _perm_mat())
    O = _ln(O, g10_ref[...], be10_ref[...])
    O = O + jnp.maximum(_dotT(O, wo1_ref[...]) + bo1_ref[...], 0.0)
    out_ref[...] = _ln(O, g11_ref[...], be11_ref[...])


def _stage5(G1, G2a, G2b, G3a, G3b, Wo1, bo1r, g10r, be10r, g11r, be11r):
    BLK = 4096
    grid = MP // BLK
    full = lambda s: pl.BlockSpec(s, lambda i: (0, 0))
    return pl.pallas_call(
        _post_body,
        grid=(grid,),
        in_specs=[pl.BlockSpec((BLK, QW), lambda i: (i, 0))] * 5 + [
            full((DIM_OUT, DIM_OUT)), full((1, DIM_OUT)),
            full((1, DIM_OUT)), full((1, DIM_OUT)), full((1, DIM_OUT)), full((1, DIM_OUT)),
        ],
        out_specs=pl.BlockSpec((BLK, DIM_OUT), lambda i: (i, 0)),
        out_shape=jax.ShapeDtypeStruct((MP, DIM_OUT), jnp.float32),
    )(G1, G2a, G2b, G3a, G3b, Wo1, bo1r, g10r, be10r, g11r, be11r)


def kernel(X, hyperedge_index, I, Wq0, bq0, Wk0, bk0, Wv0, bv0, Wo0, bo0,
           g00, be00, g01, be01, Wq1, bq1, Wk1, bk1, Wv1, bv1, Wo1, bo1,
           g10, be10, g11, be11, data, name):
    row = lambda b: b.reshape(1, -1)
    Xp = jnp.pad(X, ((0, NPAD - N), (0, 0)))
    Ipad = jnp.pad(I, ((0, 8 - NI), (0, 0)))
    nidx = jnp.pad(hyperedge_index[0], (0, MP - M))
    eidx2 = jnp.concatenate(
        [hyperedge_index[1],
         E + (jnp.arange(MP - M, dtype=jnp.int32) % 16)])
    eidx4 = jnp.pad(hyperedge_index[1], (0, MP - M))

    Ua, Ub, Qp1 = _stage1(Xp, Ipad, Wq0, row(bq0), Wk0, row(bk0),
                          Wv0, row(bv0), Wq1, row(bq1))
    zinit = jnp.zeros((ESH, UW), jnp.float32)
    Acc_a, Acc_b = _mab1_scatter(nidx, eidx2, Ua, Ub, zinit)
    Klo, Khi, Vlo, Vhi = _stage3(Acc_a, Acc_b, Ipad, Wq0, row(bq0), Wo0,
                                 row(bo0), row(g00), row(be00), row(g01),
                                 row(be01), Wk1, row(bk1), Wv1, row(bv1))
    G1, G2a, G2b, G3a, G3b = _mab2_gather(nidx, eidx4, Qp1, Klo, Khi, Vlo, Vhi)
    out = _stage5(G1, G2a, G2b, G3a, G3b, Wo1, row(bo1), row(g10), row(be10),
                  row(g11), row(be11))
    return out[:M]


# speedup vs baseline: 1.2008x; 1.0862x over previous
import functools
import math

import jax
import jax.numpy as jnp
from jax import lax
from jax.experimental import pallas as pl
from jax.experimental.pallas import tpu as pltpu
from jax.experimental.pallas import tpu_sc as plsc

N = 10000
NPAD = 10240
E = 10000
M = 160000
DIM_IN = 128
DIM_OUT = 64
HEADS = 4
DS = 16
NI = 4
UW = 144
KW = 4 * DIM_OUT

_NC = 2
_NS = 16
MP = 163840
CH = 128
TPS = MP // (CH * _NS)
TPW = MP // (CH * _NC * _NS)
ESH = E + 16
QW = 128

_SCALE = 1.0 / math.sqrt(DIM_OUT)


def _iota2(shape, dim):
    return lax.broadcasted_iota(jnp.int32, shape, dim)


def _onehot_f32(pred):
    return pred.astype(jnp.float32)


def _dotT(a, w):
    return lax.dot_general(a, w, (((1,), (1,)), ((), ())),
                           preferred_element_type=jnp.float32)


def _dot(a, w):
    return lax.dot_general(a, w, (((1,), (0,)), ((), ())),
                           preferred_element_type=jnp.float32)


def _ln(o, g, b):
    mu = jnp.mean(o, axis=1, keepdims=True)
    var = jnp.mean((o - mu) ** 2, axis=1, keepdims=True)
    return (o - mu) * lax.rsqrt(var + 1e-5) * g + b


def _perm_mat():
    i = _iota2((DIM_OUT, DIM_OUT), 0)
    j = _iota2((DIM_OUT, DIM_OUT), 1)
    return _onehot_f32(j == (i % DS) * HEADS + i // DS)


def _pre_body(x_ref, i_ref, wq0_ref, bq0_ref, wk0_ref, bk0_ref,
              wv0_ref, bv0_ref, wq1_ref, bq1_ref,
              ua_ref, ub_ref, qp1_ref):
    X = x_ref[...]
    Kp = _dotT(X, wk0_ref[...]) + bk0_ref[...]
    Vp = _dotT(X, wv0_ref[...]) + bv0_ref[...]
    Qp1 = _dotT(X, wq1_ref[...]) + bq1_ref[...]
    qp1_ref[...] = jnp.concatenate(
        [Qp1, jnp.zeros((X.shape[0], QW - DIM_OUT), jnp.float32)], axis=1)
    Qind = _dotT(i_ref[...], wq0_ref[...]) + bq0_ref[...]

    hmap = _onehot_f32(_iota2((DIM_OUT, HEADS), 0) // DS == _iota2((DIM_OUT, HEADS), 1))
    parts = []
    for inc in range(NI):
        parts.append(_dot(Kp * Qind[inc:inc + 1, :], hmap))
    S = jnp.concatenate(parts, axis=1) * _SCALE

    rows = _iota2(S.shape, 0)
    gmax = jnp.max(jnp.where(rows < N, S, -jnp.inf), axis=0, keepdims=True)
    Ex = jnp.exp(S - gmax)

    r8 = _onehot_f32(_iota2((8, 128), 1) // DS == _iota2((8, 128), 0))
    t2 = _onehot_f32(_iota2((DIM_OUT, 128), 1) % DIM_OUT == _iota2((DIM_OUT, 128), 0))
    Vt = _dot(Vp, t2)
    zpad = jnp.zeros((X.shape[0], 8), jnp.float32)
    for c, out in ((0, ua_ref), (1, ub_ref)):
        Eh = Ex[:, 8 * c:8 * c + 8]
        Eexp = _dot(Eh, r8)
        out[...] = jnp.concatenate([Eexp * Vt, Eh, zpad], axis=1)


def _stage1(Xp, Ipad, Wq0, bq0r, Wk0, bk0r, Wv0, bv0r, Wq1, bq1r):
    return pl.pallas_call(
        _pre_body,
        out_shape=[
            jax.ShapeDtypeStruct((NPAD, UW), jnp.float32),
            jax.ShapeDtypeStruct((NPAD, UW), jnp.float32),
            jax.ShapeDtypeStruct((NPAD, QW), jnp.float32),
        ],
    )(Xp, Ipad, Wq0, bq0r, Wk0, bk0r, Wv0, bv0r, Wq1, bq1r)


def _scatter_body(nidx, eidx, ua, ub, zinit, acc_a, acc_b,
                  nbuf, ebuf, rows, shared, sem):
    cid = lax.axis_index("c")
    sid = lax.axis_index("s")

    @pl.when(sid == 0)
    def _():
        pltpu.sync_copy(zinit, shared)

    plsc.subcore_barrier()

    def run(table, acc):
        def body(t, carry):
            j = sid + t * _NS

            @pl.when(j < M // CH)
            def _():
                base = j * CH
                pltpu.sync_copy(nidx.at[pl.ds(base, CH)], nbuf)
                pltpu.sync_copy(eidx.at[pl.ds(base, CH)], ebuf)
                pltpu.async_copy(table.at[nbuf], rows, sem).wait()
                pltpu.sync_copy(rows, shared.at[ebuf], add=True)

            return carry

        lax.fori_loop(0, TPS, body, 0)
        plsc.subcore_barrier()

        @pl.when(sid == 0)
        def _():
            pltpu.sync_copy(shared.at[pl.ds(0, E)], acc)

    @pl.when(cid == 0)
    def _():
        run(ua, acc_a)

    @pl.when(cid == 1)
    def _():
        run(ub, acc_b)


def _mab1_scatter(nidx, eidx, ua, ub, zinit):
    return pl.kernel(
        _scatter_body,
        out_type=[
            jax.ShapeDtypeStruct((E, UW), jnp.float32),
            jax.ShapeDtypeStruct((E, UW), jnp.float32),
        ],
        mesh=plsc.VectorSubcoreMesh(core_axis_name="c", subcore_axis_name="s"),
        compiler_params=pltpu.CompilerParams(use_tc_tiling_on_sc=False),
        scratch_types=[
            pltpu.VMEM((CH,), jnp.int32),
            pltpu.VMEM((CH,), jnp.int32),
            pltpu.VMEM((CH, UW), jnp.float32),
            pltpu.VMEM_SHARED((ESH, UW), jnp.float32),
            pltpu.SemaphoreType.DMA,
        ],
    )(nidx, eidx, ua, ub, zinit)


def _mid_body(acc_a_ref, acc_b_ref, i_ref, wq0_ref, bq0_ref,
              wo0_ref, bo0_ref, g00_ref, be00_ref, g01_ref, be01_ref,
              wk1_ref, bk1_ref, wv1_ref, bv1_ref,
              klo_ref, khi_ref, vlo_ref, vhi_ref):
    Qind = _dotT(i_ref[...], wq0_ref[...]) + bq0_ref[...]
    permM = _perm_mat()
    r4 = _onehot_f32(_iota2((HEADS, DIM_OUT), 1) // DS == _iota2((HEADS, DIM_OUT), 0))
    srcs = (acc_a_ref[...], acc_b_ref[...])
    pk, pv = [], []
    for g in range(NI):
        src = srcs[g // 2]
        lc = g % 2
        Num = src[:, DIM_OUT * lc:DIM_OUT * lc + DIM_OUT]
        Den = _dot(src[:, 128 + 4 * lc:132 + 4 * lc], r4)
        QKV = jnp.where(Den > 0, Num / Den, 0.0)
        O = QKV + Qind[g:g + 1, :]
        O = _dot(O, permM)
        O = _ln(O, g00_ref[...], be00_ref[...])
        O = O + jnp.maximum(_dotT(O, wo0_ref[...]) + bo0_ref[...], 0.0)
        O = _ln(O, g01_ref[...], be01_ref[...])
        pk.append(_dotT(O, wk1_ref[...]) + bk1_ref[...])
        pv.append(_dotT(O, wv1_ref[...]) + bv1_ref[...])
    klo_ref[...] = jnp.concatenate(pk[:2], axis=1)
    khi_ref[...] = jnp.concatenate(pk[2:], axis=1)
    vlo_ref[...] = jnp.concatenate(pv[:2], axis=1)
    vhi_ref[...] = jnp.concatenate(pv[2:], axis=1)


def _stage3(acc_a, acc_b, Ipad, Wq0, bq0r, Wo0, bo0r, g00r, be00r, g01r,
            be01r, Wk1, bk1r, Wv1, bv1r):
    BLK = 2000
    grid = E // BLK
    full = lambda s: pl.BlockSpec(s, lambda i: (0, 0))
    return pl.pallas_call(
        _mid_body,
        grid=(grid,),
        in_specs=[
            pl.BlockSpec((BLK, UW), lambda i: (i, 0)),
            pl.BlockSpec((BLK, UW), lambda i: (i, 0)),
            full((8, DIM_OUT)), full((DIM_OUT, DIM_OUT)), full((1, DIM_OUT)),
            full((DIM_OUT, DIM_OUT)), full((1, DIM_OUT)),
            full((1, DIM_OUT)), full((1, DIM_OUT)), full((1, DIM_OUT)), full((1, DIM_OUT)),
            full((DIM_OUT, DIM_OUT)), full((1, DIM_OUT)),
            full((DIM_OUT, DIM_OUT)), full((1, DIM_OUT)),
        ],
        out_specs=[pl.BlockSpec((BLK, QW), lambda i: (i, 0))] * 4,
        out_shape=[jax.ShapeDtypeStruct((E, QW), jnp.float32)] * 4,
    )(acc_a, acc_b, Ipad, Wq0, bq0r, Wo0, bo0r, g00r, be00r, g01r, be01r,
      Wk1, bk1r, Wv1, bv1r)


def _gather_body(nidx, eidx, qtab, klo, khi, vlo, vhi,
                 g1, g2a, g2b, g3a, g3b,
                 nbuf, ebuf, qrows, karows, kbrows, varows, vbrows,
                 s1, s2, s3, s4, s5):
    cid = lax.axis_index("c")
    sid = lax.axis_index("s")
    wid = sid * _NC + cid

    def body(t, carry):
        base = (wid + t * (_NC * _NS)) * CH
        pltpu.sync_copy(nidx.at[pl.ds(base, CH)], nbuf)
        pltpu.sync_copy(eidx.at[pl.ds(base, CH)], ebuf)
        c1 = pltpu.async_copy(qtab.at[nbuf], qrows, s1)
        c2 = pltpu.async_copy(klo.at[ebuf], karows, s2)
        c3 = pltpu.async_copy(khi.at[ebuf], kbrows, s3)
        c4 = pltpu.async_copy(vlo.at[ebuf], varows, s4)
        c5 = pltpu.async_copy(vhi.at[ebuf], vbrows, s5)
        c1.wait()
        c2.wait()
        c3.wait()
        c4.wait()
        c5.wait()
        pltpu.sync_copy(qrows, g1.at[pl.ds(base, CH)])
        pltpu.sync_copy(karows, g2a.at[pl.ds(base, CH)])
        pltpu.sync_copy(kbrows, g2b.at[pl.ds(base, CH)])
        pltpu.sync_copy(varows, g3a.at[pl.ds(base, CH)])
        pltpu.sync_copy(vbrows, g3b.at[pl.ds(base, CH)])
        return carry

    lax.fori_loop(0, TPW, body, 0)


def _mab2_gather(nidx, eidx, qp1, klo, khi, vlo, vhi):
    return pl.kernel(
        _gather_body,
        out_type=[jax.ShapeDtypeStruct((MP, QW), jnp.float32)] * 5,
        mesh=plsc.VectorSubcoreMesh(core_axis_name="c", subcore_axis_name="s"),
        compiler_params=pltpu.CompilerParams(use_tc_tiling_on_sc=False),
        scratch_types=[
            pltpu.VMEM((CH,), jnp.int32),
            pltpu.VMEM((CH,), jnp.int32),
            pltpu.VMEM((CH, QW), jnp.float32),
            pltpu.VMEM((CH, QW), jnp.float32),
            pltpu.VMEM((CH, QW), jnp.float32),
            pltpu.VMEM((CH, QW), jnp.float32),
            pltpu.VMEM((CH, QW), jnp.float32),
            pltpu.SemaphoreType.DMA,
            pltpu.SemaphoreType.DMA,
            pltpu.SemaphoreType.DMA,
            pltpu.SemaphoreType.DMA,
            pltpu.SemaphoreType.DMA,
        ],
    )(nidx, eidx, qp1, klo, khi, vlo, vhi)


def _post_body(g1_ref, g2a_ref, g2b_ref, g3a_ref, g3b_ref, wo1_ref, bo1_ref,
               g10_ref, be10_ref, g11_ref, be11_ref, out_ref):
    q = g1_ref[:, :DIM_OUT]
    klo = g2a_ref[...]
    khi = g2b_ref[...]
    vlo = g3a_ref[...]
    vhi = g3b_ref[...]
    t2h = _onehot_f32(_iota2((DIM_OUT, QW), 1) % DIM_OUT == _iota2((DIM_OUT, QW), 0))
    qt = _dot(q, t2h)
    cgrp = _iota2((QW, 16), 0) // DS
    ccol = _iota2((QW, 16), 1)
    msumL = _onehot_f32(ccol == cgrp)
    msumH = _onehot_f32(ccol == cgrp + 8)
    A = (_dot(qt * klo, msumL) + _dot(qt * khi, msumH)) * _SCALE
    ap = [A[:, 4 * i:4 * i + 4] for i in range(NI)]
    mx = jnp.maximum(jnp.maximum(ap[0], ap[1]), jnp.maximum(ap[2], ap[3]))
    es = [jnp.exp(p - mx) for p in ap]
    den = es[0] + es[1] + es[2] + es[3]
    w = jnp.concatenate([e / den for e in es], axis=1)
    rgrp = _iota2((16, QW), 1) // DS
    rrow = _iota2((16, QW), 0)
    r16L = _onehot_f32(rrow == rgrp)
    r16H = _onehot_f32(rrow == rgrp + 8)
    m64 = _onehot_f32(_iota2((QW, DIM_OUT), 0) % DIM_OUT == _iota2((QW, DIM_OUT), 1))
    attn = _dot(_dot(w, r16L) * vlo, m64) + _dot(_dot(w, r16H) * vhi, m64)
    O = q + attn
    O = _dot(O, _perm_mat())
    O = _ln(O, g10_ref[...], be10_ref[...])
    O = O + jnp.maximum(_dotT(O, wo1_ref[...]) + bo1_ref[...], 0.0)
    out_ref[...] = _ln(O, g11_ref[...], be11_ref[...])


def _stage5(G1, G2a, G2b, G3a, G3b, Wo1, bo1r, g10r, be10r, g11r, be11r):
    BLK = 4096
    grid = MP // BLK
    full = lambda s: pl.BlockSpec(s, lambda i: (0, 0))
    return pl.pallas_call(
        _post_body,
        grid=(grid,),
        in_specs=[pl.BlockSpec((BLK, QW), lambda i: (i, 0))] * 5 + [
            full((DIM_OUT, DIM_OUT)), full((1, DIM_OUT)),
            full((1, DIM_OUT)), full((1, DIM_OUT)), full((1, DIM_OUT)), full((1, DIM_OUT)),
        ],
        out_specs=pl.BlockSpec((BLK, DIM_OUT), lambda i: (i, 0)),
        out_shape=jax.ShapeDtypeStruct((MP, DIM_OUT), jnp.float32),
    )(G1, G2a, G2b, G3a, G3b, Wo1, bo1r, g10r, be10r, g11r, be11r)


def kernel(X, hyperedge_index, I, Wq0, bq0, Wk0, bk0, Wv0, bv0, Wo0, bo0,
           g00, be00, g01, be01, Wq1, bq1, Wk1, bk1, Wv1, bv1, Wo1, bo1,
           g10, be10, g11, be11, data, name):
    row = lambda b: b.reshape(1, -1)
    Xp = jnp.pad(X, ((0, NPAD - N), (0, 0)))
    Ipad = jnp.pad(I, ((0, 8 - NI), (0, 0)))
    nidx = jnp.pad(hyperedge_index[0], (0, MP - M))
    eidx2 = jnp.concatenate(
        [hyperedge_index[1],
         E + (jnp.arange(MP - M, dtype=jnp.int32) % 16)])
    eidx4 = jnp.pad(hyperedge_index[1], (0, MP - M))

    Ua, Ub, Qp1 = _stage1(Xp, Ipad, Wq0, row(bq0), Wk0, row(bk0),
                          Wv0, row(bv0), Wq1, row(bq1))
    zinit = jnp.zeros((ESH, UW), jnp.float32)
    Acc_a, Acc_b = _mab1_scatter(nidx, eidx2, Ua, Ub, zinit)
    Klo, Khi, Vlo, Vhi = _stage3(Acc_a, Acc_b, Ipad, Wq0, row(bq0), Wo0,
                                 row(bo0), row(g00), row(be00), row(g01),
                                 row(be01), Wk1, row(bk1), Wv1, row(bv1))
    G1, G2a, G2b, G3a, G3b = _mab2_gather(nidx, eidx4, Qp1, Klo, Khi, Vlo, Vhi)
    out = _stage5(G1, G2a, G2b, G3a, G3b, Wo1, row(bo1), row(g10), row(be10),
                  row(g11), row(be11))
    return out[:M]


# speedup vs baseline: 1.3310x; 1.1084x over previous
import functools
import math

import jax
import jax.numpy as jnp
from jax import lax
from jax.experimental import pallas as pl
from jax.experimental.pallas import tpu as pltpu
from jax.experimental.pallas import tpu_sc as plsc

N = 10000
NPAD = 10240
E = 10000
M = 160000
DIM_IN = 128
DIM_OUT = 64
HEADS = 4
DS = 16
NI = 4
UW = 144
KW = 4 * DIM_OUT

_NC = 2
_NS = 16
MP = 163840
CH = 128
TPS = MP // (CH * _NS)
TPW = MP // (CH * _NC * _NS)
ESH = E + 16
QW = 128

_SCALE = 1.0 / math.sqrt(DIM_OUT)


def _iota2(shape, dim):
    return lax.broadcasted_iota(jnp.int32, shape, dim)


def _onehot_f32(pred):
    return pred.astype(jnp.float32)


def _dotT(a, w):
    return lax.dot_general(a, w, (((1,), (1,)), ((), ())),
                           preferred_element_type=jnp.float32)


def _dot(a, w):
    return lax.dot_general(a, w, (((1,), (0,)), ((), ())),
                           preferred_element_type=jnp.float32)


def _ln(o, g, b):
    mu = jnp.mean(o, axis=1, keepdims=True)
    var = jnp.mean((o - mu) ** 2, axis=1, keepdims=True)
    return (o - mu) * lax.rsqrt(var + 1e-5) * g + b


def _perm_mat():
    i = _iota2((DIM_OUT, DIM_OUT), 0)
    j = _iota2((DIM_OUT, DIM_OUT), 1)
    return _onehot_f32(j == (i % DS) * HEADS + i // DS)


def _pre_body(x_ref, i_ref, wq0_ref, bq0_ref, wk0_ref, bk0_ref,
              wv0_ref, bv0_ref, wq1_ref, bq1_ref,
              ua_ref, ub_ref, qp1_ref):
    X = x_ref[...]
    Kp = _dotT(X, wk0_ref[...]) + bk0_ref[...]
    Vp = _dotT(X, wv0_ref[...]) + bv0_ref[...]
    Qp1 = _dotT(X, wq1_ref[...]) + bq1_ref[...]
    qp1_ref[...] = jnp.concatenate(
        [Qp1, jnp.zeros((X.shape[0], QW - DIM_OUT), jnp.float32)], axis=1)
    Qind = _dotT(i_ref[...], wq0_ref[...]) + bq0_ref[...]

    hmap = _onehot_f32(_iota2((DIM_OUT, HEADS), 0) // DS == _iota2((DIM_OUT, HEADS), 1))
    parts = []
    for inc in range(NI):
        parts.append(_dot(Kp * Qind[inc:inc + 1, :], hmap))
    S = jnp.concatenate(parts, axis=1) * _SCALE

    rows = _iota2(S.shape, 0)
    gmax = jnp.max(jnp.where(rows < N, S, -jnp.inf), axis=0, keepdims=True)
    Ex = jnp.exp(S - gmax)

    r8 = _onehot_f32(_iota2((8, 128), 1) // DS == _iota2((8, 128), 0))
    t2 = _onehot_f32(_iota2((DIM_OUT, 128), 1) % DIM_OUT == _iota2((DIM_OUT, 128), 0))
    Vt = _dot(Vp, t2)
    zpad = jnp.zeros((X.shape[0], 8), jnp.float32)
    for c, out in ((0, ua_ref), (1, ub_ref)):
        Eh = Ex[:, 8 * c:8 * c + 8]
        Eexp = _dot(Eh, r8)
        out[...] = jnp.concatenate([Eexp * Vt, Eh, zpad], axis=1)


def _stage1(Xp, Ipad, Wq0, bq0r, Wk0, bk0r, Wv0, bv0r, Wq1, bq1r):
    return pl.pallas_call(
        _pre_body,
        out_shape=[
            jax.ShapeDtypeStruct((NPAD, UW), jnp.float32),
            jax.ShapeDtypeStruct((NPAD, UW), jnp.float32),
            jax.ShapeDtypeStruct((NPAD, QW), jnp.float32),
        ],
    )(Xp, Ipad, Wq0, bq0r, Wk0, bk0r, Wv0, bv0r, Wq1, bq1r)


def _scatter_body(nidx, eidx, ua, ub, zinit, acc_a, acc_b,
                  nbuf, ebuf, rows, shared, sem):
    cid = lax.axis_index("c")
    sid = lax.axis_index("s")

    @pl.when(sid == 0)
    def _():
        pltpu.sync_copy(zinit, shared)

    plsc.subcore_barrier()

    def run(table, acc):
        def body(t, carry):
            j = sid + t * _NS

            @pl.when(j < M // CH)
            def _():
                base = j * CH
                pltpu.sync_copy(nidx.at[pl.ds(base, CH)], nbuf)
                pltpu.sync_copy(eidx.at[pl.ds(base, CH)], ebuf)
                pltpu.async_copy(table.at[nbuf], rows, sem).wait()
                pltpu.sync_copy(rows, shared.at[ebuf], add=True)

            return carry

        lax.fori_loop(0, TPS, body, 0)
        plsc.subcore_barrier()

        @pl.when(sid == 0)
        def _():
            pltpu.sync_copy(shared.at[pl.ds(0, E)], acc)

    @pl.when(cid == 0)
    def _():
        run(ua, acc_a)

    @pl.when(cid == 1)
    def _():
        run(ub, acc_b)


def _mab1_scatter(nidx, eidx, ua, ub, zinit):
    return pl.kernel(
        _scatter_body,
        out_type=[
            jax.ShapeDtypeStruct((E, UW), jnp.float32),
            jax.ShapeDtypeStruct((E, UW), jnp.float32),
        ],
        mesh=plsc.VectorSubcoreMesh(core_axis_name="c", subcore_axis_name="s"),
        compiler_params=pltpu.CompilerParams(use_tc_tiling_on_sc=False),
        scratch_types=[
            pltpu.VMEM((CH,), jnp.int32),
            pltpu.VMEM((CH,), jnp.int32),
            pltpu.VMEM((CH, UW), jnp.float32),
            pltpu.VMEM_SHARED((ESH, UW), jnp.float32),
            pltpu.SemaphoreType.DMA,
        ],
    )(nidx, eidx, ua, ub, zinit)


def _mid_body(acc_a_ref, acc_b_ref, i_ref, wq0_ref, bq0_ref,
              wo0_ref, bo0_ref, g00_ref, be00_ref, g01_ref, be01_ref,
              wk1_ref, bk1_ref, wv1_ref, bv1_ref,
              klo_ref, khi_ref, vlo_ref, vhi_ref):
    Qind = _dotT(i_ref[...], wq0_ref[...]) + bq0_ref[...]
    permM = _perm_mat()
    r4 = _onehot_f32(_iota2((HEADS, DIM_OUT), 1) // DS == _iota2((HEADS, DIM_OUT), 0))
    srcs = (acc_a_ref[...], acc_b_ref[...])
    pk, pv = [], []
    for g in range(NI):
        src = srcs[g // 2]
        lc = g % 2
        Num = src[:, DIM_OUT * lc:DIM_OUT * lc + DIM_OUT]
        Den = _dot(src[:, 128 + 4 * lc:132 + 4 * lc], r4)
        QKV = jnp.where(Den > 0, Num / Den, 0.0)
        O = QKV + Qind[g:g + 1, :]
        O = _dot(O, permM)
        O = _ln(O, g00_ref[...], be00_ref[...])
        O = O + jnp.maximum(_dotT(O, wo0_ref[...]) + bo0_ref[...], 0.0)
        O = _ln(O, g01_ref[...], be01_ref[...])
        pk.append(_dotT(O, wk1_ref[...]) + bk1_ref[...])
        pv.append(_dotT(O, wv1_ref[...]) + bv1_ref[...])
    klo_ref[...] = jnp.concatenate(pk[:2], axis=1)
    khi_ref[...] = jnp.concatenate(pk[2:], axis=1)
    vlo_ref[...] = jnp.concatenate(pv[:2], axis=1)
    vhi_ref[...] = jnp.concatenate(pv[2:], axis=1)


def _stage3(acc_a, acc_b, Ipad, Wq0, bq0r, Wo0, bo0r, g00r, be00r, g01r,
            be01r, Wk1, bk1r, Wv1, bv1r):
    BLK = 2000
    grid = E // BLK
    full = lambda s: pl.BlockSpec(s, lambda i: (0, 0))
    return pl.pallas_call(
        _mid_body,
        grid=(grid,),
        in_specs=[
            pl.BlockSpec((BLK, UW), lambda i: (i, 0)),
            pl.BlockSpec((BLK, UW), lambda i: (i, 0)),
            full((8, DIM_OUT)), full((DIM_OUT, DIM_OUT)), full((1, DIM_OUT)),
            full((DIM_OUT, DIM_OUT)), full((1, DIM_OUT)),
            full((1, DIM_OUT)), full((1, DIM_OUT)), full((1, DIM_OUT)), full((1, DIM_OUT)),
            full((DIM_OUT, DIM_OUT)), full((1, DIM_OUT)),
            full((DIM_OUT, DIM_OUT)), full((1, DIM_OUT)),
        ],
        out_specs=[pl.BlockSpec((BLK, QW), lambda i: (i, 0))] * 4,
        out_shape=[jax.ShapeDtypeStruct((E, QW), jnp.float32)] * 4,
    )(acc_a, acc_b, Ipad, Wq0, bq0r, Wo0, bo0r, g00r, be00r, g01r, be01r,
      Wk1, bk1r, Wv1, bv1r)


HALF = MP // 2
HCHUNK = HALF // CH
TPH = HCHUNK // (_NC * _NS)


def _make_gather_body(off):
    def _gather_body(nidx, eidx, qtab, klo, khi, vlo, vhi,
                     g1, g2a, g2b, g3a, g3b,
                     nbuf, ebuf, qrows, karows, kbrows, varows, vbrows,
                     s1, s2, s3, s4, s5):
        cid = lax.axis_index("c")
        sid = lax.axis_index("s")
        wid = sid * _NC + cid

        def body(t, carry):
            base = (wid + t * (_NC * _NS)) * CH
            src = off + base
            pltpu.sync_copy(nidx.at[pl.ds(src, CH)], nbuf)
            pltpu.sync_copy(eidx.at[pl.ds(src, CH)], ebuf)
            c1 = pltpu.async_copy(qtab.at[nbuf], qrows, s1)
            c2 = pltpu.async_copy(klo.at[ebuf], karows, s2)
            c3 = pltpu.async_copy(khi.at[ebuf], kbrows, s3)
            c4 = pltpu.async_copy(vlo.at[ebuf], varows, s4)
            c5 = pltpu.async_copy(vhi.at[ebuf], vbrows, s5)
            c1.wait()
            c2.wait()
            c3.wait()
            c4.wait()
            c5.wait()
            pltpu.sync_copy(qrows, g1.at[pl.ds(base, CH)])
            pltpu.sync_copy(karows, g2a.at[pl.ds(base, CH)])
            pltpu.sync_copy(kbrows, g2b.at[pl.ds(base, CH)])
            pltpu.sync_copy(varows, g3a.at[pl.ds(base, CH)])
            pltpu.sync_copy(vbrows, g3b.at[pl.ds(base, CH)])
            return carry

        lax.fori_loop(0, TPH, body, 0)

    return _gather_body


def _mab2_gather(nidx, eidx, qp1, klo, khi, vlo, vhi, half):
    return pl.kernel(
        _make_gather_body(half * HALF),
        out_type=[jax.ShapeDtypeStruct((HALF, QW), jnp.float32)] * 5,
        mesh=plsc.VectorSubcoreMesh(core_axis_name="c", subcore_axis_name="s"),
        compiler_params=pltpu.CompilerParams(use_tc_tiling_on_sc=False),
        scratch_types=[
            pltpu.VMEM((CH,), jnp.int32),
            pltpu.VMEM((CH,), jnp.int32),
            pltpu.VMEM((CH, QW), jnp.float32),
            pltpu.VMEM((CH, QW), jnp.float32),
            pltpu.VMEM((CH, QW), jnp.float32),
            pltpu.VMEM((CH, QW), jnp.float32),
            pltpu.VMEM((CH, QW), jnp.float32),
            pltpu.SemaphoreType.DMA,
            pltpu.SemaphoreType.DMA,
            pltpu.SemaphoreType.DMA,
            pltpu.SemaphoreType.DMA,
            pltpu.SemaphoreType.DMA,
        ],
    )(nidx, eidx, qp1, klo, khi, vlo, vhi)


def _post_body(g1_ref, g2a_ref, g2b_ref, g3a_ref, g3b_ref, wo1_ref, bo1_ref,
               g10_ref, be10_ref, g11_ref, be11_ref, out_ref):
    q = g1_ref[:, :DIM_OUT]
    klo = g2a_ref[...]
    khi = g2b_ref[...]
    vlo = g3a_ref[...]
    vhi = g3b_ref[...]
    t2h = _onehot_f32(_iota2((DIM_OUT, QW), 1) % DIM_OUT == _iota2((DIM_OUT, QW), 0))
    qt = _dot(q, t2h)
    cgrp = _iota2((QW, 16), 0) // DS
    ccol = _iota2((QW, 16), 1)
    msumL = _onehot_f32(ccol == cgrp)
    msumH = _onehot_f32(ccol == cgrp + 8)
    A = (_dot(qt * klo, msumL) + _dot(qt * khi, msumH)) * _SCALE
    ap = [A[:, 4 * i:4 * i + 4] for i in range(NI)]
    mx = jnp.maximum(jnp.maximum(ap[0], ap[1]), jnp.maximum(ap[2], ap[3]))
    es = [jnp.exp(p - mx) for p in ap]
    den = es[0] + es[1] + es[2] + es[3]
    w = jnp.concatenate([e / den for e in es], axis=1)
    rgrp = _iota2((16, QW), 1) // DS
    rrow = _iota2((16, QW), 0)
    r16L = _onehot_f32(rrow == rgrp)
    r16H = _onehot_f32(rrow == rgrp + 8)
    m64 = _onehot_f32(_iota2((QW, DIM_OUT), 0) % DIM_OUT == _iota2((QW, DIM_OUT), 1))
    attn = _dot(_dot(w, r16L) * vlo, m64) + _dot(_dot(w, r16H) * vhi, m64)
    O = q + attn
    O = _dot(O, _perm_mat())
    O = _ln(O, g10_ref[...], be10_ref[...])
    O = O + jnp.maximum(_dotT(O, wo1_ref[...]) + bo1_ref[...], 0.0)
    out_ref[...] = _ln(O, g11_ref[...], be11_ref[...])


def _stage5(G1, G2a, G2b, G3a, G3b, Wo1, bo1r, g10r, be10r, g11r, be11r):
    BLK = 4096
    grid = HALF // BLK
    full = lambda s: pl.BlockSpec(s, lambda i: (0, 0))
    return pl.pallas_call(
        _post_body,
        grid=(grid,),
        in_specs=[pl.BlockSpec((BLK, QW), lambda i: (i, 0))] * 5 + [
            full((DIM_OUT, DIM_OUT)), full((1, DIM_OUT)),
            full((1, DIM_OUT)), full((1, DIM_OUT)), full((1, DIM_OUT)), full((1, DIM_OUT)),
        ],
        out_specs=pl.BlockSpec((BLK, DIM_OUT), lambda i: (i, 0)),
        out_shape=jax.ShapeDtypeStruct((HALF, DIM_OUT), jnp.float32),
    )(G1, G2a, G2b, G3a, G3b, Wo1, bo1r, g10r, be10r, g11r, be11r)


def kernel(X, hyperedge_index, I, Wq0, bq0, Wk0, bk0, Wv0, bv0, Wo0, bo0,
           g00, be00, g01, be01, Wq1, bq1, Wk1, bk1, Wv1, bv1, Wo1, bo1,
           g10, be10, g11, be11, data, name):
    row = lambda b: b.reshape(1, -1)
    Xp = jnp.pad(X, ((0, NPAD - N), (0, 0)))
    Ipad = jnp.pad(I, ((0, 8 - NI), (0, 0)))
    nidx = jnp.pad(hyperedge_index[0], (0, MP - M))
    eidx2 = jnp.concatenate(
        [hyperedge_index[1],
         E + (jnp.arange(MP - M, dtype=jnp.int32) % 16)])
    eidx4 = jnp.pad(hyperedge_index[1], (0, MP - M))

    Ua, Ub, Qp1 = _stage1(Xp, Ipad, Wq0, row(bq0), Wk0, row(bk0),
                          Wv0, row(bv0), Wq1, row(bq1))
    zinit = jnp.zeros((ESH, UW), jnp.float32)
    Acc_a, Acc_b = _mab1_scatter(nidx, eidx2, Ua, Ub, zinit)
    Klo, Khi, Vlo, Vhi = _stage3(Acc_a, Acc_b, Ipad, Wq0, row(bq0), Wo0,
                                 row(bo0), row(g00), row(be00), row(g01),
                                 row(be01), Wk1, row(bk1), Wv1, row(bv1))
    outs = []
    for half in (0, 1):
        Gs = _mab2_gather(nidx, eidx4, Qp1, Klo, Khi, Vlo, Vhi, half)
        outs.append(_stage5(*Gs, Wo1, row(bo1), row(g10), row(be10),
                            row(g11), row(be11)))
    return jnp.concatenate(outs, axis=0)[:M]


# speedup vs baseline: 1.3871x; 1.0422x over previous
import functools
import math

import jax
import jax.numpy as jnp
from jax import lax
from jax.experimental import pallas as pl
from jax.experimental.pallas import tpu as pltpu
from jax.experimental.pallas import tpu_sc as plsc

N = 10000
NPAD = 10240
E = 10000
M = 160000
DIM_IN = 128
DIM_OUT = 64
HEADS = 4
DS = 16
NI = 4
UW = 144
KW = 4 * DIM_OUT

_NC = 2
_NS = 16
MP = 163840
CH = 128
TPS = MP // (CH * _NS)
TPW = MP // (CH * _NC * _NS)
ESH = E + 16
QW = 128

_SCALE = 1.0 / math.sqrt(DIM_OUT)


def _iota2(shape, dim):
    return lax.broadcasted_iota(jnp.int32, shape, dim)


def _onehot_f32(pred):
    return pred.astype(jnp.float32)


def _dotT(a, w):
    return lax.dot_general(a, w, (((1,), (1,)), ((), ())),
                           preferred_element_type=jnp.float32)


def _dot(a, w):
    return lax.dot_general(a, w, (((1,), (0,)), ((), ())),
                           preferred_element_type=jnp.float32)


def _ln(o, g, b):
    mu = jnp.mean(o, axis=1, keepdims=True)
    var = jnp.mean((o - mu) ** 2, axis=1, keepdims=True)
    return (o - mu) * lax.rsqrt(var + 1e-5) * g + b


def _perm_mat():
    i = _iota2((DIM_OUT, DIM_OUT), 0)
    j = _iota2((DIM_OUT, DIM_OUT), 1)
    return _onehot_f32(j == (i % DS) * HEADS + i // DS)


def _pre_body(x_ref, i_ref, wq0_ref, bq0_ref, wk0_ref, bk0_ref,
              wv0_ref, bv0_ref, wq1_ref, bq1_ref,
              ua_ref, ub_ref, qp1_ref):
    X = x_ref[...]
    Kp = _dotT(X, wk0_ref[...]) + bk0_ref[...]
    Vp = _dotT(X, wv0_ref[...]) + bv0_ref[...]
    Qp1 = _dotT(X, wq1_ref[...]) + bq1_ref[...]
    qp1_ref[...] = jnp.concatenate(
        [Qp1, jnp.zeros((X.shape[0], QW - DIM_OUT), jnp.float32)], axis=1)
    Qind = _dotT(i_ref[...], wq0_ref[...]) + bq0_ref[...]

    hmap = _onehot_f32(_iota2((DIM_OUT, HEADS), 0) // DS == _iota2((DIM_OUT, HEADS), 1))
    parts = []
    for inc in range(NI):
        parts.append(_dot(Kp * Qind[inc:inc + 1, :], hmap))
    S = jnp.concatenate(parts, axis=1) * _SCALE

    rows = _iota2(S.shape, 0)
    gmax = jnp.max(jnp.where(rows < N, S, -jnp.inf), axis=0, keepdims=True)
    Ex = jnp.exp(S - gmax)

    r8 = _onehot_f32(_iota2((8, 128), 1) // DS == _iota2((8, 128), 0))
    t2 = _onehot_f32(_iota2((DIM_OUT, 128), 1) % DIM_OUT == _iota2((DIM_OUT, 128), 0))
    Vt = _dot(Vp, t2)
    zpad = jnp.zeros((X.shape[0], 8), jnp.float32)
    for c, out in ((0, ua_ref), (1, ub_ref)):
        Eh = Ex[:, 8 * c:8 * c + 8]
        Eexp = _dot(Eh, r8)
        out[...] = jnp.concatenate([Eexp * Vt, Eh, zpad], axis=1)


def _stage1(Xp, Ipad, Wq0, bq0r, Wk0, bk0r, Wv0, bv0r, Wq1, bq1r):
    return pl.pallas_call(
        _pre_body,
        out_shape=[
            jax.ShapeDtypeStruct((NPAD, UW), jnp.float32),
            jax.ShapeDtypeStruct((NPAD, UW), jnp.float32),
            jax.ShapeDtypeStruct((NPAD, QW), jnp.float32),
        ],
    )(Xp, Ipad, Wq0, bq0r, Wk0, bk0r, Wv0, bv0r, Wq1, bq1r)


def _scatter_body(nidx, eidx, ua, ub, zinit, acc_a, acc_b,
                  nbuf, ebuf, rows, shared, sem):
    cid = lax.axis_index("c")
    sid = lax.axis_index("s")

    @pl.when(sid == 0)
    def _():
        pltpu.sync_copy(zinit, shared)

    plsc.subcore_barrier()

    def run(table, acc):
        def body(t, carry):
            j = sid + t * _NS

            @pl.when(j < M // CH)
            def _():
                base = j * CH
                pltpu.sync_copy(nidx.at[pl.ds(base, CH)], nbuf)
                pltpu.sync_copy(eidx.at[pl.ds(base, CH)], ebuf)
                pltpu.async_copy(table.at[nbuf], rows, sem).wait()
                pltpu.sync_copy(rows, shared.at[ebuf], add=True)

            return carry

        lax.fori_loop(0, TPS, body, 0)
        plsc.subcore_barrier()

        @pl.when(sid == 0)
        def _():
            pltpu.sync_copy(shared.at[pl.ds(0, E)], acc)

    @pl.when(cid == 0)
    def _():
        run(ua, acc_a)

    @pl.when(cid == 1)
    def _():
        run(ub, acc_b)


def _mab1_scatter(nidx, eidx, ua, ub, zinit):
    return pl.kernel(
        _scatter_body,
        out_type=[
            jax.ShapeDtypeStruct((E, UW), jnp.float32),
            jax.ShapeDtypeStruct((E, UW), jnp.float32),
        ],
        mesh=plsc.VectorSubcoreMesh(core_axis_name="c", subcore_axis_name="s"),
        compiler_params=pltpu.CompilerParams(use_tc_tiling_on_sc=False),
        scratch_types=[
            pltpu.VMEM((CH,), jnp.int32),
            pltpu.VMEM((CH,), jnp.int32),
            pltpu.VMEM((CH, UW), jnp.float32),
            pltpu.VMEM_SHARED((ESH, UW), jnp.float32),
            pltpu.SemaphoreType.DMA,
        ],
    )(nidx, eidx, ua, ub, zinit)


def _mid_body(acc_a_ref, acc_b_ref, i_ref, wq0_ref, bq0_ref,
              wo0_ref, bo0_ref, g00_ref, be00_ref, g01_ref, be01_ref,
              wk1_ref, bk1_ref, wv1_ref, bv1_ref,
              klo_ref, khi_ref, vlo_ref, vhi_ref):
    Qind = _dotT(i_ref[...], wq0_ref[...]) + bq0_ref[...]
    permM = _perm_mat()
    r4 = _onehot_f32(_iota2((HEADS, DIM_OUT), 1) // DS == _iota2((HEADS, DIM_OUT), 0))
    srcs = (acc_a_ref[...], acc_b_ref[...])
    pk, pv = [], []
    for g in range(NI):
        src = srcs[g // 2]
        lc = g % 2
        Num = src[:, DIM_OUT * lc:DIM_OUT * lc + DIM_OUT]
        Den = _dot(src[:, 128 + 4 * lc:132 + 4 * lc], r4)
        QKV = jnp.where(Den > 0, Num / Den, 0.0)
        O = QKV + Qind[g:g + 1, :]
        O = _dot(O, permM)
        O = _ln(O, g00_ref[...], be00_ref[...])
        O = O + jnp.maximum(_dotT(O, wo0_ref[...]) + bo0_ref[...], 0.0)
        O = _ln(O, g01_ref[...], be01_ref[...])
        pk.append(_dotT(O, wk1_ref[...]) + bk1_ref[...])
        pv.append(_dotT(O, wv1_ref[...]) + bv1_ref[...])
    klo_ref[...] = jnp.concatenate(pk[:2], axis=1)
    khi_ref[...] = jnp.concatenate(pk[2:], axis=1)
    vlo_ref[...] = jnp.concatenate(pv[:2], axis=1)
    vhi_ref[...] = jnp.concatenate(pv[2:], axis=1)


def _stage3(acc_a, acc_b, Ipad, Wq0, bq0r, Wo0, bo0r, g00r, be00r, g01r,
            be01r, Wk1, bk1r, Wv1, bv1r):
    BLK = 2000
    grid = E // BLK
    full = lambda s: pl.BlockSpec(s, lambda i: (0, 0))
    return pl.pallas_call(
        _mid_body,
        grid=(grid,),
        in_specs=[
            pl.BlockSpec((BLK, UW), lambda i: (i, 0)),
            pl.BlockSpec((BLK, UW), lambda i: (i, 0)),
            full((8, DIM_OUT)), full((DIM_OUT, DIM_OUT)), full((1, DIM_OUT)),
            full((DIM_OUT, DIM_OUT)), full((1, DIM_OUT)),
            full((1, DIM_OUT)), full((1, DIM_OUT)), full((1, DIM_OUT)), full((1, DIM_OUT)),
            full((DIM_OUT, DIM_OUT)), full((1, DIM_OUT)),
            full((DIM_OUT, DIM_OUT)), full((1, DIM_OUT)),
        ],
        out_specs=[pl.BlockSpec((BLK, QW), lambda i: (i, 0))] * 4,
        out_shape=[jax.ShapeDtypeStruct((E, QW), jnp.float32)] * 4,
    )(acc_a, acc_b, Ipad, Wq0, bq0r, Wo0, bo0r, g00r, be00r, g01r, be01r,
      Wk1, bk1r, Wv1, bv1r)


NSPLIT = 4
HALF = MP // NSPLIT
HCHUNK = HALF // CH
TPH = HCHUNK // (_NC * _NS)


def _make_gather_body(off):
    def _gather_body(nidx, eidx, qtab, klo, khi, vlo, vhi,
                     g1, g2a, g2b, g3a, g3b,
                     nbuf, ebuf, qrows, karows, kbrows, varows, vbrows,
                     s1, s2, s3, s4, s5):
        cid = lax.axis_index("c")
        sid = lax.axis_index("s")
        wid = sid * _NC + cid

        def body(t, carry):
            base = (wid + t * (_NC * _NS)) * CH
            src = off + base
            pltpu.sync_copy(nidx.at[pl.ds(src, CH)], nbuf)
            pltpu.sync_copy(eidx.at[pl.ds(src, CH)], ebuf)
            c1 = pltpu.async_copy(qtab.at[nbuf], qrows, s1)
            c2 = pltpu.async_copy(klo.at[ebuf], karows, s2)
            c3 = pltpu.async_copy(khi.at[ebuf], kbrows, s3)
            c4 = pltpu.async_copy(vlo.at[ebuf], varows, s4)
            c5 = pltpu.async_copy(vhi.at[ebuf], vbrows, s5)
            c1.wait()
            c2.wait()
            c3.wait()
            c4.wait()
            c5.wait()
            pltpu.sync_copy(qrows, g1.at[pl.ds(base, CH)])
            pltpu.sync_copy(karows, g2a.at[pl.ds(base, CH)])
            pltpu.sync_copy(kbrows, g2b.at[pl.ds(base, CH)])
            pltpu.sync_copy(varows, g3a.at[pl.ds(base, CH)])
            pltpu.sync_copy(vbrows, g3b.at[pl.ds(base, CH)])
            return carry

        lax.fori_loop(0, TPH, body, 0)

    return _gather_body


def _mab2_gather(nidx, eidx, qp1, klo, khi, vlo, vhi, half):
    return pl.kernel(
        _make_gather_body(half * HALF),
        out_type=[jax.ShapeDtypeStruct((HALF, QW), jnp.float32)] * 5,
        mesh=plsc.VectorSubcoreMesh(core_axis_name="c", subcore_axis_name="s"),
        compiler_params=pltpu.CompilerParams(use_tc_tiling_on_sc=False),
        scratch_types=[
            pltpu.VMEM((CH,), jnp.int32),
            pltpu.VMEM((CH,), jnp.int32),
            pltpu.VMEM((CH, QW), jnp.float32),
            pltpu.VMEM((CH, QW), jnp.float32),
            pltpu.VMEM((CH, QW), jnp.float32),
            pltpu.VMEM((CH, QW), jnp.float32),
            pltpu.VMEM((CH, QW), jnp.float32),
            pltpu.SemaphoreType.DMA,
            pltpu.SemaphoreType.DMA,
            pltpu.SemaphoreType.DMA,
            pltpu.SemaphoreType.DMA,
            pltpu.SemaphoreType.DMA,
        ],
    )(nidx, eidx, qp1, klo, khi, vlo, vhi)


def _post_body(g1_ref, g2a_ref, g2b_ref, g3a_ref, g3b_ref, wo1_ref, bo1_ref,
               g10_ref, be10_ref, g11_ref, be11_ref, out_ref):
    q = g1_ref[:, :DIM_OUT]
    klo = g2a_ref[...]
    khi = g2b_ref[...]
    vlo = g3a_ref[...]
    vhi = g3b_ref[...]
    t2h = _onehot_f32(_iota2((DIM_OUT, QW), 1) % DIM_OUT == _iota2((DIM_OUT, QW), 0))
    qt = _dot(q, t2h)
    cgrp = _iota2((QW, 16), 0) // DS
    ccol = _iota2((QW, 16), 1)
    msumL = _onehot_f32(ccol == cgrp)
    msumH = _onehot_f32(ccol == cgrp + 8)
    A = (_dot(qt * klo, msumL) + _dot(qt * khi, msumH)) * _SCALE
    ap = [A[:, 4 * i:4 * i + 4] for i in range(NI)]
    mx = jnp.maximum(jnp.maximum(ap[0], ap[1]), jnp.maximum(ap[2], ap[3]))
    es = [jnp.exp(p - mx) for p in ap]
    den = es[0] + es[1] + es[2] + es[3]
    w = jnp.concatenate([e / den for e in es], axis=1)
    rgrp = _iota2((16, QW), 1) // DS
    rrow = _iota2((16, QW), 0)
    r16L = _onehot_f32(rrow == rgrp)
    r16H = _onehot_f32(rrow == rgrp + 8)
    m64 = _onehot_f32(_iota2((QW, DIM_OUT), 0) % DIM_OUT == _iota2((QW, DIM_OUT), 1))
    attn = _dot(_dot(w, r16L) * vlo, m64) + _dot(_dot(w, r16H) * vhi, m64)
    O = q + attn
    O = _dot(O, _perm_mat())
    O = _ln(O, g10_ref[...], be10_ref[...])
    O = O + jnp.maximum(_dotT(O, wo1_ref[...]) + bo1_ref[...], 0.0)
    out_ref[...] = _ln(O, g11_ref[...], be11_ref[...])


def _stage5(G1, G2a, G2b, G3a, G3b, Wo1, bo1r, g10r, be10r, g11r, be11r):
    BLK = 4096
    grid = HALF // BLK
    full = lambda s: pl.BlockSpec(s, lambda i: (0, 0))
    return pl.pallas_call(
        _post_body,
        grid=(grid,),
        in_specs=[pl.BlockSpec((BLK, QW), lambda i: (i, 0))] * 5 + [
            full((DIM_OUT, DIM_OUT)), full((1, DIM_OUT)),
            full((1, DIM_OUT)), full((1, DIM_OUT)), full((1, DIM_OUT)), full((1, DIM_OUT)),
        ],
        out_specs=pl.BlockSpec((BLK, DIM_OUT), lambda i: (i, 0)),
        out_shape=jax.ShapeDtypeStruct((HALF, DIM_OUT), jnp.float32),
    )(G1, G2a, G2b, G3a, G3b, Wo1, bo1r, g10r, be10r, g11r, be11r)


def kernel(X, hyperedge_index, I, Wq0, bq0, Wk0, bk0, Wv0, bv0, Wo0, bo0,
           g00, be00, g01, be01, Wq1, bq1, Wk1, bk1, Wv1, bv1, Wo1, bo1,
           g10, be10, g11, be11, data, name):
    row = lambda b: b.reshape(1, -1)
    Xp = jnp.pad(X, ((0, NPAD - N), (0, 0)))
    Ipad = jnp.pad(I, ((0, 8 - NI), (0, 0)))
    nidx = jnp.pad(hyperedge_index[0], (0, MP - M))
    eidx2 = jnp.concatenate(
        [hyperedge_index[1],
         E + (jnp.arange(MP - M, dtype=jnp.int32) % 16)])
    eidx4 = jnp.pad(hyperedge_index[1], (0, MP - M))

    Ua, Ub, Qp1 = _stage1(Xp, Ipad, Wq0, row(bq0), Wk0, row(bk0),
                          Wv0, row(bv0), Wq1, row(bq1))
    zinit = jnp.zeros((ESH, UW), jnp.float32)
    Acc_a, Acc_b = _mab1_scatter(nidx, eidx2, Ua, Ub, zinit)
    Klo, Khi, Vlo, Vhi = _stage3(Acc_a, Acc_b, Ipad, Wq0, row(bq0), Wo0,
                                 row(bo0), row(g00), row(be00), row(g01),
                                 row(be01), Wk1, row(bk1), Wv1, row(bv1))
    outs = []
    for half in range(NSPLIT):
        Gs = _mab2_gather(nidx, eidx4, Qp1, Klo, Khi, Vlo, Vhi, half)
        outs.append(_stage5(*Gs, Wo1, row(bo1), row(g10), row(be10),
                            row(g11), row(be11)))
    return jnp.concatenate(outs, axis=0)[:M]
